# baseline (speedup 1.0000x reference)
import functools

import jax
import jax.numpy as jnp
from jax import lax
from jax.experimental import pallas as pl
from jax.experimental.pallas import tpu as pltpu
from jax.experimental.pallas import tpu_sc as plsc

N = 10000
NP = 10240
E = 320000
EP = 331776
D = 128
K = 10
G = 64

NC = 2
NS = 16
CH = 128
ET = EP // (NC * NS)
NCHUNK = ET // CH
RPT = NP // NS

_MESH = plsc.VectorSubcoreMesh(core_axis_name="c", subcore_axis_name="s")

NB = 4
NCROW = EP // CH
CPT = NCHUNK


@functools.partial(
    pl.kernel,
    out_type=jax.ShapeDtypeStruct((NC, NP), jnp.float32),
    mesh=_MESH,
    scratch_types=[
        pltpu.VMEM((CPT, 2, CH), jnp.int32),
        pltpu.VMEM((CH,), jnp.float32),
        pltpu.VMEM_SHARED((NP,), jnp.float32),
        pltpu.SemaphoreType.DMA((9,)),
    ],
)
def _sc_degree(e2_hbm, ones_hbm, zeros1_hbm, deg_hbm, ev, onesv, acc, ssem):
    c = lax.axis_index("c")
    s = lax.axis_index("s")
    pltpu.sync_copy(zeros1_hbm, acc.at[pl.ds(s * RPT, RPT)])
    pltpu.sync_copy(ones_hbm, onesv)
    cbase = c * (NCROW // NC) + s * CPT
    pltpu.sync_copy(e2_hbm.at[pl.ds(cbase, CPT)], ev)
    plsc.subcore_barrier()

    def step(j, carry):
        for u in range(9):
            i = j * 9 + u

            @pl.when(i >= 9)
            def _():
                pltpu.make_async_copy(onesv, acc.at[ev.at[0, 1]],
                                      ssem.at[u]).wait()

            pltpu.async_copy(onesv, acc.at[ev.at[i, 1]], ssem.at[u],
                             add=True)
        return carry

    lax.fori_loop(0, CPT // 9, step, 0)
    for u in range(9):
        pltpu.make_async_copy(onesv, acc.at[ev.at[0, 1]], ssem.at[u]).wait()
    plsc.subcore_barrier()
    pltpu.sync_copy(acc.at[pl.ds(s * RPT, RPT)], deg_hbm.at[c, pl.ds(s * RPT, RPT)])


UNROLL = 9
DH = D // 2
NBUF = 3
ZR = 32


@functools.partial(
    pl.kernel,
    out_type=jax.ShapeDtypeStruct((2, NC, NP, DH), jnp.float32),
    mesh=_MESH,
    scratch_types=[
        pltpu.VMEM((CPT, 2, CH), jnp.int32),
        pltpu.VMEM((NBUF, CH, DH), jnp.float32),
        pltpu.VMEM((ZR, DH), jnp.float32),
        pltpu.VMEM_SHARED((NP, DH), jnp.float32),
        pltpu.VMEM_SHARED((NP, DH), jnp.float32),
        pltpu.SemaphoreType.DMA((NBUF,)),
        pltpu.SemaphoreType.DMA((NBUF,)),
    ],
    compiler_params=pltpu.CompilerParams(use_tc_tiling_on_sc=False),
)
def _sc_spmm(ya_hbm, yb_hbm, e2_hbm, zeros_hbm, s_hbm,
             idxa, bufs, zv, ysh, acc, gsem, ssem):
    c = lax.axis_index("c")
    s = lax.axis_index("s")
    cbase = c * (NCROW // NC) + s * CPT
    pltpu.sync_copy(zeros_hbm, zv)
    pltpu.sync_copy(e2_hbm.at[pl.ds(cbase, CPT)], idxa)

    def wait_gather(p):
        pltpu.make_async_copy(ysh.at[idxa.at[0, 0]], bufs.at[p],
                              gsem.at[p]).wait()

    def wait_scatter(p):
        pltpu.make_async_copy(bufs.at[p], acc.at[idxa.at[0, 1]],
                              ssem.at[p]).wait()

    for h, y_hbm in ((0, ya_hbm), (1, yb_hbm)):
        pltpu.sync_copy(y_hbm.at[pl.ds(s * RPT, RPT)],
                        ysh.at[pl.ds(s * RPT, RPT)])
        for z in range(RPT // ZR):
            pltpu.sync_copy(zv, acc.at[pl.ds(s * RPT + z * ZR, ZR)])
        plsc.subcore_barrier()
        pltpu.async_copy(ysh.at[idxa.at[0, 0]], bufs.at[0], gsem.at[0])

        def step(j, carry):
            for u in range(UNROLL):
                i = j * UNROLL + u
                p = u % NBUF

                @pl.when(i >= 2)
                def _():
                    wait_scatter((u + 1) % NBUF)

                @pl.when(i + 1 < CPT)
                def _():
                    pltpu.async_copy(ysh.at[idxa.at[i + 1, 0]],
                                     bufs.at[(u + 1) % NBUF],
                                     gsem.at[(u + 1) % NBUF])

                wait_gather(p)
                pltpu.async_copy(bufs.at[p], acc.at[idxa.at[i, 1]],
                                 ssem.at[p], add=True)
            return carry

        lax.fori_loop(0, CPT // UNROLL, step, 0)
        wait_scatter((CPT - 2) % NBUF)
        wait_scatter((CPT - 1) % NBUF)
        plsc.subcore_barrier()
        pltpu.sync_copy(acc.at[pl.ds(s * RPT, RPT)],
                        s_hbm.at[h, c, pl.ds(s * RPT, RPT)])
        plsc.subcore_barrier()


_BLK = 2048


def _tc_prep_body(dg0_ref, dg1_ref, x_ref, dv_ref, ya_ref, yb_ref):
    deg = dg0_ref[...] + dg1_ref[...]
    dinv = jnp.where(deg > 0.0,
                     1.0 / jnp.sqrt(jnp.maximum(deg, 1e-12)), 0.0)
    dv = jnp.broadcast_to(dinv, x_ref.shape)
    dv_ref[...] = dv
    y = dv * x_ref[...]
    ya_ref[...] = y[:, :DH]
    yb_ref[...] = y[:, DH:]


def _tc_prep(dg0, dg1, x):
    nb = NP // _BLK
    return pl.pallas_call(
        _tc_prep_body,
        grid=(nb,),
        in_specs=[
            pl.BlockSpec((_BLK, 1), lambda i: (i, 0)),
            pl.BlockSpec((_BLK, 1), lambda i: (i, 0)),
            pl.BlockSpec((_BLK, D), lambda i: (i, 0)),
        ],
        out_specs=[
            pl.BlockSpec((_BLK, D), lambda i: (i, 0)),
            pl.BlockSpec((_BLK, DH), lambda i: (i, 0)),
            pl.BlockSpec((_BLK, DH), lambda i: (i, 0)),
        ],
        out_shape=[
            jax.ShapeDtypeStruct((NP, D), jnp.float32),
            jax.ShapeDtypeStruct((NP, DH), jnp.float32),
            jax.ShapeDtypeStruct((NP, DH), jnp.float32),
        ],
    )(dg0, dg1, x)


def _tc_combine_first_body(s_ref, dv_ref, tx_ref, ya_ref, yb_ref):
    dv = dv_ref[...]
    ssum = jnp.concatenate(
        [s_ref[0, 0] + s_ref[0, 1], s_ref[1, 0] + s_ref[1, 1]], axis=1)
    tx = -1.0 * dv * ssum
    tx_ref[...] = tx
    y = dv * tx
    ya_ref[...] = y[:, :DH]
    yb_ref[...] = y[:, DH:]


def _tc_combine_body(s_ref, dv_ref, txp_ref, tx_ref, ya_ref, yb_ref):
    dv = dv_ref[...]
    ssum = jnp.concatenate(
        [s_ref[0, 0] + s_ref[0, 1], s_ref[1, 0] + s_ref[1, 1]], axis=1)
    tx = -2.0 * dv * ssum - txp_ref[...]
    tx_ref[...] = tx
    y = dv * tx
    ya_ref[...] = y[:, :DH]
    yb_ref[...] = y[:, DH:]


def _tc_combine(s, dv, txp=None):
    nb = NP // _BLK
    in_specs = [
        pl.BlockSpec((2, NC, _BLK, DH), lambda i: (0, 0, i, 0)),
        pl.BlockSpec((_BLK, D), lambda i: (i, 0)),
    ]
    args = [s, dv]
    body = _tc_combine_first_body
    if txp is not None:
        in_specs.append(pl.BlockSpec((_BLK, D), lambda i: (i, 0)))
        args.append(txp)
        body = _tc_combine_body
    return pl.pallas_call(
        body,
        grid=(nb,),
        in_specs=in_specs,
        out_specs=[
            pl.BlockSpec((_BLK, D), lambda i: (i, 0)),
            pl.BlockSpec((_BLK, DH), lambda i: (i, 0)),
            pl.BlockSpec((_BLK, DH), lambda i: (i, 0)),
        ],
        out_shape=[
            jax.ShapeDtypeStruct((NP, D), jnp.float32),
            jax.ShapeDtypeStruct((NP, DH), jnp.float32),
            jax.ShapeDtypeStruct((NP, DH), jnp.float32),
        ],
    )(*args)


_FBLK = 512
_FNB = NP // _FBLK


def _tc_final_body(*refs):
    tx_refs = refs[:K]
    w_ref, b_ref, batch_ref = refs[K], refs[K + 1], refs[K + 2]
    out_ref = refs[K + 3]
    acc_ref, cnt_ref = refs[K + 4], refs[K + 5]
    i = pl.program_id(0)

    h = jnp.zeros((_FBLK, D), jnp.float32)
    for k in range(K):
        h = h + jnp.dot(tx_refs[k][...], w_ref[k],
                        preferred_element_type=jnp.float32)
    h = h + b_ref[...]

    bb = batch_ref[...]
    gid = lax.broadcasted_iota(jnp.int32, (G, _FBLK), 0)
    oht = (jnp.broadcast_to(bb, (G, _FBLK)) == gid).astype(jnp.float32)
    contrib = jnp.dot(oht, h, preferred_element_type=jnp.float32)
    csum = jnp.sum(oht, axis=1, keepdims=True)
    cnt_c = jnp.broadcast_to(csum, (G, D))

    @pl.when(i == 0)
    def _():
        acc_ref[...] = jnp.zeros((G, D), jnp.float32)
        cnt_ref[...] = jnp.zeros((G, D), jnp.float32)

    acc_ref[...] += contrib
    cnt_ref[...] += cnt_c

    @pl.when(i == _FNB - 1)
    def _():
        out_ref[...] = acc_ref[...] / jnp.maximum(cnt_ref[...], 1.0)


def _tc_final(txs, W, b2, batch2):
    in_specs = [pl.BlockSpec((_FBLK, D), lambda i: (i, 0)) for _ in range(K)]
    in_specs += [
        pl.BlockSpec((K, D, D), lambda i: (0, 0, 0)),
        pl.BlockSpec((1, D), lambda i: (0, 0)),
        pl.BlockSpec((1, _FBLK), lambda i: (0, i)),
    ]
    return pl.pallas_call(
        _tc_final_body,
        grid=(_FNB,),
        in_specs=in_specs,
        out_specs=pl.BlockSpec((G, D), lambda i: (0, 0)),
        out_shape=jax.ShapeDtypeStruct((G, D), jnp.float32),
        scratch_shapes=[
            pltpu.VMEM((G, D), jnp.float32),
            pltpu.VMEM((G, D), jnp.float32),
        ],
    )(*txs, W, b2, batch2)


def kernel(x, edge_index, batch, W, b):
    xp = jnp.zeros((NP, D), jnp.float32).at[:N].set(x)
    npad = EP - E
    row = jnp.concatenate(
        [edge_index[0], (N + (jnp.arange(npad, dtype=jnp.int32) % 16))]
    ).reshape(NCROW, CH)
    col = jnp.concatenate(
        [edge_index[1], jnp.zeros((npad,), jnp.int32)]).reshape(NCROW, CH)
    e2 = jnp.stack([col, row], axis=1)
    batch2 = jnp.full((1, NP), G, jnp.int32).at[0, :N].set(batch)

    ones_ch = jnp.ones((CH,), jnp.float32)
    zeros1 = jnp.zeros((RPT,), jnp.float32)
    zeros2 = jnp.zeros((ZR, DH), jnp.float32)
    b2 = b.reshape(1, D)

    deg = _sc_degree(e2, ones_ch, zeros1)
    dg0 = deg[0].reshape(NP, 1)
    dg1 = deg[1].reshape(NP, 1)
    dv, ya, yb = _tc_prep(dg0, dg1, xp)

    txs = [xp]
    txp = None
    for _ in range(1, K):
        s = _sc_spmm(ya, yb, e2, zeros2)
        tx, ya, yb = _tc_combine(s, dv, txp)
        txp = txs[-1]
        txs.append(tx)

    return _tc_final(txs, W, b2, batch2)


# speedup vs baseline: 1.1715x; 1.1715x over previous
import functools

import jax
import jax.numpy as jnp
from jax import lax
from jax.experimental import pallas as pl
from jax.experimental.pallas import tpu as pltpu
from jax.experimental.pallas import tpu_sc as plsc

N = 10000
NP = 10240
E = 320000
EP = 327680
D = 128
K = 10
G = 64

NC = 2
NS = 16
CH = 128
ET = EP // (NC * NS)
NCHUNK = ET // CH
RPT = NP // NS

_MESH = plsc.VectorSubcoreMesh(core_axis_name="c", subcore_axis_name="s")

NB = 4
NCROW = EP // CH
CPT = NCHUNK


@functools.partial(
    pl.kernel,
    out_type=jax.ShapeDtypeStruct((NC, NP), jnp.float32),
    mesh=_MESH,
    scratch_types=[
        pltpu.VMEM((CPT, 2, CH), jnp.int32),
        pltpu.VMEM((CH,), jnp.float32),
        pltpu.VMEM_SHARED((NP,), jnp.float32),
        pltpu.SemaphoreType.DMA((8,)),
    ],
)
def _sc_degree(e2_hbm, ones_hbm, zeros1_hbm, deg_hbm, ev, onesv, acc, ssem):
    c = lax.axis_index("c")
    s = lax.axis_index("s")
    pltpu.sync_copy(zeros1_hbm, acc.at[pl.ds(s * RPT, RPT)])
    pltpu.sync_copy(ones_hbm, onesv)
    cbase = c * (NCROW // NC) + s * CPT
    pltpu.sync_copy(e2_hbm.at[pl.ds(cbase, CPT)], ev)
    plsc.subcore_barrier()

    def step(j, carry):
        for u in range(8):
            i = j * 8 + u

            @pl.when(i >= 8)
            def _():
                pltpu.make_async_copy(onesv, acc.at[ev.at[0, 1]],
                                      ssem.at[u]).wait()

            pltpu.async_copy(onesv, acc.at[ev.at[i, 1]], ssem.at[u],
                             add=True)
        return carry

    lax.fori_loop(0, CPT // 8, step, 0)
    for u in range(8):
        pltpu.make_async_copy(onesv, acc.at[ev.at[0, 1]], ssem.at[u]).wait()
    plsc.subcore_barrier()
    pltpu.sync_copy(acc.at[pl.ds(s * RPT, RPT)], deg_hbm.at[c, pl.ds(s * RPT, RPT)])


RNG = 10
UNROLL = 10
DH = D // 2
NBUF = 5
ZR = 64


@functools.partial(
    pl.kernel,
    out_type=jax.ShapeDtypeStruct((NC, NP, DH), jnp.float32),
    mesh=_MESH,
    scratch_types=[
        pltpu.VMEM((RNG, 2, CH), jnp.int32),
        pltpu.VMEM((NBUF, CH, DH), jnp.float32),
        pltpu.VMEM((ZR, DH), jnp.float32),
        pltpu.VMEM_SHARED((NP, DH), jnp.float32),
        pltpu.VMEM_SHARED((NP, DH), jnp.float32),
        pltpu.SemaphoreType.DMA((RNG,)),
        pltpu.SemaphoreType.DMA((NBUF,)),
        pltpu.SemaphoreType.DMA((NBUF,)),
    ],
    compiler_params=pltpu.CompilerParams(use_tc_tiling_on_sc=False),
)
def _sc_spmm(y_hbm, e2_hbm, zeros_hbm, s_hbm,
             idxr, bufs, zv, ysh, acc, isem, gsem, ssem):
    c = lax.axis_index("c")
    s = lax.axis_index("s")
    cbase = c * (NCROW // NC) + s * CPT
    pltpu.sync_copy(zeros_hbm, zv)

    def load_idx(ci, q):
        pltpu.async_copy(e2_hbm.at[cbase + ci], idxr.at[q], isem.at[q])

    def wait_idx(q):
        pltpu.make_async_copy(e2_hbm.at[cbase], idxr.at[q], isem.at[q]).wait()

    def wait_gather(p):
        pltpu.make_async_copy(ysh.at[idxr.at[0, 0]], bufs.at[p],
                              gsem.at[p]).wait()

    def wait_scatter(p):
        pltpu.make_async_copy(bufs.at[p], acc.at[idxr.at[0, 1]],
                              ssem.at[p]).wait()

    pltpu.sync_copy(y_hbm.at[pl.ds(s * RPT, RPT)],
                    ysh.at[pl.ds(s * RPT, RPT)])
    for z in range(RPT // ZR):
        pltpu.sync_copy(zv, acc.at[pl.ds(s * RPT + z * ZR, ZR)])
    for q in range(4):
        load_idx(q, q)
    plsc.subcore_barrier()
    wait_idx(0)
    wait_idx(1)
    pltpu.async_copy(ysh.at[idxr.at[0, 0]], bufs.at[0], gsem.at[0])
    pltpu.async_copy(ysh.at[idxr.at[1, 0]], bufs.at[1], gsem.at[1])

    def step(j, carry):
        for u in range(UNROLL):
            i = j * UNROLL + u
            p = u % NBUF

            @pl.when(i + 2 < CPT)
            def _():
                wait_idx((u + 2) % RNG)

            @pl.when(i >= 3)
            def _():
                wait_scatter((u + 2) % NBUF)

            @pl.when(i + 2 < CPT)
            def _():
                pltpu.async_copy(ysh.at[idxr.at[(u + 2) % RNG, 0]],
                                 bufs.at[(u + 2) % NBUF],
                                 gsem.at[(u + 2) % NBUF])

            @pl.when(i + 4 < CPT)
            def _():
                load_idx(i + 4, (u + 4) % RNG)

            wait_gather(p)
            pltpu.async_copy(bufs.at[p], acc.at[idxr.at[u % RNG, 1]],
                             ssem.at[p], add=True)
        return carry

    lax.fori_loop(0, CPT // UNROLL, step, 0)
    for t in (3, 2, 1):
        wait_scatter((CPT - t) % NBUF)
    plsc.subcore_barrier()
    pltpu.sync_copy(acc.at[pl.ds(s * RPT, RPT)],
                    s_hbm.at[c, pl.ds(s * RPT, RPT)])


_BLK = 2048


def _tc_prep_body(dg0_ref, dg1_ref, x_ref, dva_ref, dvb_ref, ya_ref, yb_ref):
    deg = dg0_ref[...] + dg1_ref[...]
    dinv = jnp.where(deg > 0.0,
                     1.0 / jnp.sqrt(jnp.maximum(deg, 1e-12)), 0.0)
    dv = jnp.broadcast_to(dinv, x_ref.shape)
    y = dv * x_ref[...]
    dva_ref[...] = dv[:, :DH]
    dvb_ref[...] = dv[:, DH:]
    ya_ref[...] = y[:, :DH]
    yb_ref[...] = y[:, DH:]


def _tc_prep(dg0, dg1, x):
    nb = NP // _BLK
    half = [
        pl.BlockSpec((_BLK, DH), lambda i: (i, 0)),
        jax.ShapeDtypeStruct((NP, DH), jnp.float32),
    ]
    return pl.pallas_call(
        _tc_prep_body,
        grid=(nb,),
        in_specs=[
            pl.BlockSpec((_BLK, 1), lambda i: (i, 0)),
            pl.BlockSpec((_BLK, 1), lambda i: (i, 0)),
            pl.BlockSpec((_BLK, D), lambda i: (i, 0)),
        ],
        out_specs=[half[0]] * 4,
        out_shape=[half[1]] * 4,
    )(dg0, dg1, x)


def _tc_combine_first_body(s_ref, dv_ref, tx_ref, y_ref):
    dv = dv_ref[...]
    tx = -1.0 * dv * (s_ref[0] + s_ref[1])
    tx_ref[...] = tx
    y_ref[...] = dv * tx


def _tc_combine_body(s_ref, dv_ref, txp_ref, tx_ref, y_ref):
    dv = dv_ref[...]
    tx = -2.0 * dv * (s_ref[0] + s_ref[1]) - txp_ref[...]
    tx_ref[...] = tx
    y_ref[...] = dv * tx


def _tc_combine(s, dv, txp=None):
    nb = NP // _BLK
    in_specs = [
        pl.BlockSpec((NC, _BLK, DH), lambda i: (0, i, 0)),
        pl.BlockSpec((_BLK, DH), lambda i: (i, 0)),
    ]
    args = [s, dv]
    body = _tc_combine_first_body
    if txp is not None:
        in_specs.append(pl.BlockSpec((_BLK, DH), lambda i: (i, 0)))
        args.append(txp)
        body = _tc_combine_body
    return pl.pallas_call(
        body,
        grid=(nb,),
        in_specs=in_specs,
        out_specs=[
            pl.BlockSpec((_BLK, DH), lambda i: (i, 0)),
            pl.BlockSpec((_BLK, DH), lambda i: (i, 0)),
        ],
        out_shape=[
            jax.ShapeDtypeStruct((NP, DH), jnp.float32),
            jax.ShapeDtypeStruct((NP, DH), jnp.float32),
        ],
    )(*args)


_FBLK = 512
_FNB = NP // _FBLK


def _tc_final_body(*refs):
    tx_refs = refs[:2 * K]
    w_ref, b_ref, batch_ref = refs[2 * K], refs[2 * K + 1], refs[2 * K + 2]
    out_ref = refs[2 * K + 3]
    acc_ref, cnt_ref = refs[2 * K + 4], refs[2 * K + 5]
    i = pl.program_id(0)

    h = jnp.zeros((_FBLK, D), jnp.float32)
    for k in range(K):
        h = h + jnp.dot(tx_refs[2 * k][...], w_ref[k, :DH, :],
                        preferred_element_type=jnp.float32)
        h = h + jnp.dot(tx_refs[2 * k + 1][...], w_ref[k, DH:, :],
                        preferred_element_type=jnp.float32)
    h = h + b_ref[...]

    bb = batch_ref[...]
    gid = lax.broadcasted_iota(jnp.int32, (G, _FBLK), 0)
    oht = (jnp.broadcast_to(bb, (G, _FBLK)) == gid).astype(jnp.float32)
    contrib = jnp.dot(oht, h, preferred_element_type=jnp.float32)
    csum = jnp.sum(oht, axis=1, keepdims=True)
    cnt_c = jnp.broadcast_to(csum, (G, D))

    @pl.when(i == 0)
    def _():
        acc_ref[...] = jnp.zeros((G, D), jnp.float32)
        cnt_ref[...] = jnp.zeros((G, D), jnp.float32)

    acc_ref[...] += contrib
    cnt_ref[...] += cnt_c

    @pl.when(i == _FNB - 1)
    def _():
        out_ref[...] = acc_ref[...] / jnp.maximum(cnt_ref[...], 1.0)


def _tc_final(txs, W, b2, batch2):
    in_specs = [pl.BlockSpec((_FBLK, DH), lambda i: (i, 0))
                for _ in range(2 * K)]
    in_specs += [
        pl.BlockSpec((K, D, D), lambda i: (0, 0, 0)),
        pl.BlockSpec((1, D), lambda i: (0, 0)),
        pl.BlockSpec((1, _FBLK), lambda i: (0, i)),
    ]
    return pl.pallas_call(
        _tc_final_body,
        grid=(_FNB,),
        in_specs=in_specs,
        out_specs=pl.BlockSpec((G, D), lambda i: (0, 0)),
        out_shape=jax.ShapeDtypeStruct((G, D), jnp.float32),
        scratch_shapes=[
            pltpu.VMEM((G, D), jnp.float32),
            pltpu.VMEM((G, D), jnp.float32),
        ],
    )(*txs, W, b2, batch2)


def kernel(x, edge_index, batch, W, b):
    xp = jnp.zeros((NP, D), jnp.float32).at[:N].set(x)
    npad = EP - E
    row = jnp.concatenate(
        [edge_index[0], (N + (jnp.arange(npad, dtype=jnp.int32) % 16))]
    ).reshape(NCROW, CH)
    col = jnp.concatenate(
        [edge_index[1], jnp.zeros((npad,), jnp.int32)]).reshape(NCROW, CH)
    e2 = jnp.stack([col, row], axis=1)
    batch2 = jnp.full((1, NP), G, jnp.int32).at[0, :N].set(batch)

    ones_ch = jnp.ones((CH,), jnp.float32)
    zeros1 = jnp.zeros((RPT,), jnp.float32)
    zeros2 = jnp.zeros((ZR, DH), jnp.float32)
    b2 = b.reshape(1, D)

    deg = _sc_degree(e2, ones_ch, zeros1)
    dg0 = deg[0].reshape(NP, 1)
    dg1 = deg[1].reshape(NP, 1)
    dva, dvb, ya, yb = _tc_prep(dg0, dg1, xp)

    txs = [xp[:, :DH], xp[:, DH:]]
    txpa = txpb = None
    for _ in range(1, K):
        sa = _sc_spmm(ya, e2, zeros2)
        sb = _sc_spmm(yb, e2, zeros2)
        txa, ya = _tc_combine(sa, dva, txpa)
        txb, yb = _tc_combine(sb, dvb, txpb)
        txpa, txpb = txs[-2], txs[-1]
        txs.extend([txa, txb])

    return _tc_final(txs, W, b2, batch2)


# speedup vs baseline: 1.2045x; 1.0282x over previous
import functools

import jax
import jax.numpy as jnp
from jax import lax
from jax.experimental import pallas as pl
from jax.experimental.pallas import tpu as pltpu
from jax.experimental.pallas import tpu_sc as plsc

N = 10000
NP = 10240
E = 320000
EP = 327680
D = 128
K = 10
G = 64

NC = 2
NS = 16
CH = 128
ET = EP // (NC * NS)
NCHUNK = ET // CH
RPT = NP // NS

_MESH = plsc.VectorSubcoreMesh(core_axis_name="c", subcore_axis_name="s")

NB = 4
NCROW = EP // CH
CPT = NCHUNK


@functools.partial(
    pl.kernel,
    out_type=jax.ShapeDtypeStruct((NC, NP), jnp.float32),
    mesh=_MESH,
    scratch_types=[
        pltpu.VMEM((CPT, 2, CH), jnp.int32),
        pltpu.VMEM((CH,), jnp.float32),
        pltpu.VMEM_SHARED((NP,), jnp.float32),
        pltpu.SemaphoreType.DMA((8,)),
    ],
)
def _sc_degree(e2_hbm, ones_hbm, zeros1_hbm, deg_hbm, ev, onesv, acc, ssem):
    c = lax.axis_index("c")
    s = lax.axis_index("s")
    pltpu.sync_copy(zeros1_hbm, acc.at[pl.ds(s * RPT, RPT)])
    pltpu.sync_copy(ones_hbm, onesv)
    cbase = c * (NCROW // NC) + s * CPT
    pltpu.sync_copy(e2_hbm.at[pl.ds(cbase, CPT)], ev)
    plsc.subcore_barrier()

    def step(j, carry):
        for u in range(8):
            i = j * 8 + u

            @pl.when(i >= 8)
            def _():
                pltpu.make_async_copy(onesv, acc.at[ev.at[0, 1]],
                                      ssem.at[u]).wait()

            pltpu.async_copy(onesv, acc.at[ev.at[i, 1]], ssem.at[u],
                             add=True)
        return carry

    lax.fori_loop(0, CPT // 8, step, 0)
    for u in range(8):
        pltpu.make_async_copy(onesv, acc.at[ev.at[0, 1]], ssem.at[u]).wait()
    plsc.subcore_barrier()
    pltpu.sync_copy(acc.at[pl.ds(s * RPT, RPT)], deg_hbm.at[c, pl.ds(s * RPT, RPT)])


RNG = 10
UNROLL = 10
DH = D // 2
NBUF = 5
ZR = 64


@functools.partial(
    pl.kernel,
    out_type=jax.ShapeDtypeStruct((NC, NP, DH), jnp.float32),
    mesh=_MESH,
    scratch_types=[
        pltpu.VMEM((RNG, 2, CH), jnp.int32),
        pltpu.VMEM((NBUF, CH, DH), jnp.float32),
        pltpu.VMEM((ZR, DH), jnp.float32),
        pltpu.VMEM_SHARED((NP, DH), jnp.float32),
        pltpu.VMEM_SHARED((NP, DH), jnp.float32),
        pltpu.SemaphoreType.DMA((RNG,)),
        pltpu.SemaphoreType.DMA((NBUF,)),
        pltpu.SemaphoreType.DMA((NBUF,)),
    ],
    compiler_params=pltpu.CompilerParams(use_tc_tiling_on_sc=False),
)
def _sc_spmm(y_hbm, e2_hbm, zeros_hbm, s_hbm,
             idxr, bufs, zv, ysh, acc, isem, gsem, ssem):
    c = lax.axis_index("c")
    s = lax.axis_index("s")
    cbase = c * (NCROW // NC) + s * CPT
    pltpu.sync_copy(zeros_hbm, zv)

    def load_idx(ci, q):
        pltpu.async_copy(e2_hbm.at[cbase + ci], idxr.at[q], isem.at[q])

    def wait_idx(q):
        pltpu.make_async_copy(e2_hbm.at[cbase], idxr.at[q], isem.at[q]).wait()

    def wait_gather(p):
        pltpu.make_async_copy(ysh.at[idxr.at[0, 0]], bufs.at[p],
                              gsem.at[p]).wait()

    def wait_scatter(p):
        pltpu.make_async_copy(bufs.at[p], acc.at[idxr.at[0, 1]],
                              ssem.at[p]).wait()

    pltpu.async_copy(y_hbm.at[pl.ds(s * RPT, RPT)],
                     ysh.at[pl.ds(s * RPT, RPT)], gsem.at[4])
    for z in range(RPT // ZR):
        pltpu.async_copy(zv, acc.at[pl.ds(s * RPT + z * ZR, ZR)],
                         ssem.at[z % NBUF])
    for q in range(4):
        load_idx(q, q)
    for z in range(RPT // ZR):
        pltpu.make_async_copy(
            zv, acc.at[pl.ds(s * RPT, ZR)], ssem.at[z % NBUF]).wait()
    pltpu.make_async_copy(y_hbm.at[pl.ds(s * RPT, RPT)],
                          ysh.at[pl.ds(s * RPT, RPT)], gsem.at[4]).wait()
    plsc.subcore_barrier()
    wait_idx(0)
    wait_idx(1)
    pltpu.async_copy(ysh.at[idxr.at[0, 0]], bufs.at[0], gsem.at[0])
    pltpu.async_copy(ysh.at[idxr.at[1, 0]], bufs.at[1], gsem.at[1])

    def step(j, carry):
        for u in range(UNROLL):
            i = j * UNROLL + u
            p = u % NBUF

            @pl.when(i + 2 < CPT)
            def _():
                wait_idx((u + 2) % RNG)

            @pl.when(i >= 3)
            def _():
                wait_scatter((u + 2) % NBUF)

            @pl.when(i + 2 < CPT)
            def _():
                pltpu.async_copy(ysh.at[idxr.at[(u + 2) % RNG, 0]],
                                 bufs.at[(u + 2) % NBUF],
                                 gsem.at[(u + 2) % NBUF])

            @pl.when(i + 4 < CPT)
            def _():
                load_idx(i + 4, (u + 4) % RNG)

            wait_gather(p)
            pltpu.async_copy(bufs.at[p], acc.at[idxr.at[u % RNG, 1]],
                             ssem.at[p], add=True)
        return carry

    lax.fori_loop(0, CPT // UNROLL, step, 0)
    for t in (3, 2, 1):
        wait_scatter((CPT - t) % NBUF)
    plsc.subcore_barrier()
    pltpu.sync_copy(acc.at[pl.ds(s * RPT, RPT)],
                    s_hbm.at[c, pl.ds(s * RPT, RPT)])


_BLK = 2048


def _tc_prep_body(dg0_ref, dg1_ref, x_ref, dva_ref, dvb_ref, ya_ref, yb_ref):
    deg = dg0_ref[...] + dg1_ref[...]
    dinv = jnp.where(deg > 0.0,
                     1.0 / jnp.sqrt(jnp.maximum(deg, 1e-12)), 0.0)
    dv = jnp.broadcast_to(dinv, x_ref.shape)
    y = dv * x_ref[...]
    dva_ref[...] = dv[:, :DH]
    dvb_ref[...] = dv[:, DH:]
    ya_ref[...] = y[:, :DH]
    yb_ref[...] = y[:, DH:]


def _tc_prep(dg0, dg1, x):
    nb = NP // _BLK
    half = [
        pl.BlockSpec((_BLK, DH), lambda i: (i, 0)),
        jax.ShapeDtypeStruct((NP, DH), jnp.float32),
    ]
    return pl.pallas_call(
        _tc_prep_body,
        grid=(nb,),
        in_specs=[
            pl.BlockSpec((_BLK, 1), lambda i: (i, 0)),
            pl.BlockSpec((_BLK, 1), lambda i: (i, 0)),
            pl.BlockSpec((_BLK, D), lambda i: (i, 0)),
        ],
        out_specs=[half[0]] * 4,
        out_shape=[half[1]] * 4,
    )(dg0, dg1, x)


def _tc_combine_first_body(s_ref, dv_ref, tx_ref, y_ref):
    dv = dv_ref[...]
    tx = -1.0 * dv * (s_ref[0] + s_ref[1])
    tx_ref[...] = tx
    y_ref[...] = dv * tx


def _tc_combine_body(s_ref, dv_ref, txp_ref, tx_ref, y_ref):
    dv = dv_ref[...]
    tx = -2.0 * dv * (s_ref[0] + s_ref[1]) - txp_ref[...]
    tx_ref[...] = tx
    y_ref[...] = dv * tx


def _tc_combine(s, dv, txp=None):
    nb = NP // _BLK
    in_specs = [
        pl.BlockSpec((NC, _BLK, DH), lambda i: (0, i, 0)),
        pl.BlockSpec((_BLK, DH), lambda i: (i, 0)),
    ]
    args = [s, dv]
    body = _tc_combine_first_body
    if txp is not None:
        in_specs.append(pl.BlockSpec((_BLK, DH), lambda i: (i, 0)))
        args.append(txp)
        body = _tc_combine_body
    return pl.pallas_call(
        body,
        grid=(nb,),
        in_specs=in_specs,
        out_specs=[
            pl.BlockSpec((_BLK, DH), lambda i: (i, 0)),
            pl.BlockSpec((_BLK, DH), lambda i: (i, 0)),
        ],
        out_shape=[
            jax.ShapeDtypeStruct((NP, DH), jnp.float32),
            jax.ShapeDtypeStruct((NP, DH), jnp.float32),
        ],
    )(*args)


_FBLK = 512
_FNB = NP // _FBLK


def _tc_final_body(*refs):
    tx_refs = refs[:2 * K]
    w_ref, b_ref, batch_ref = refs[2 * K], refs[2 * K + 1], refs[2 * K + 2]
    out_ref = refs[2 * K + 3]
    acc_ref, cnt_ref = refs[2 * K + 4], refs[2 * K + 5]
    i = pl.program_id(0)

    h = jnp.zeros((_FBLK, D), jnp.float32)
    for k in range(K):
        h = h + jnp.dot(tx_refs[2 * k][...], w_ref[k, :DH, :],
                        preferred_element_type=jnp.float32)
        h = h + jnp.dot(tx_refs[2 * k + 1][...], w_ref[k, DH:, :],
                        preferred_element_type=jnp.float32)
    h = h + b_ref[...]

    bb = batch_ref[...]
    gid = lax.broadcasted_iota(jnp.int32, (G, _FBLK), 0)
    oht = (jnp.broadcast_to(bb, (G, _FBLK)) == gid).astype(jnp.float32)
    contrib = jnp.dot(oht, h, preferred_element_type=jnp.float32)
    csum = jnp.sum(oht, axis=1, keepdims=True)
    cnt_c = jnp.broadcast_to(csum, (G, D))

    @pl.when(i == 0)
    def _():
        acc_ref[...] = jnp.zeros((G, D), jnp.float32)
        cnt_ref[...] = jnp.zeros((G, D), jnp.float32)

    acc_ref[...] += contrib
    cnt_ref[...] += cnt_c

    @pl.when(i == _FNB - 1)
    def _():
        out_ref[...] = acc_ref[...] / jnp.maximum(cnt_ref[...], 1.0)


def _tc_final(txs, W, b2, batch2):
    in_specs = [pl.BlockSpec((_FBLK, DH), lambda i: (i, 0))
                for _ in range(2 * K)]
    in_specs += [
        pl.BlockSpec((K, D, D), lambda i: (0, 0, 0)),
        pl.BlockSpec((1, D), lambda i: (0, 0)),
        pl.BlockSpec((1, _FBLK), lambda i: (0, i)),
    ]
    return pl.pallas_call(
        _tc_final_body,
        grid=(_FNB,),
        in_specs=in_specs,
        out_specs=pl.BlockSpec((G, D), lambda i: (0, 0)),
        out_shape=jax.ShapeDtypeStruct((G, D), jnp.float32),
        scratch_shapes=[
            pltpu.VMEM((G, D), jnp.float32),
            pltpu.VMEM((G, D), jnp.float32),
        ],
    )(*txs, W, b2, batch2)


def kernel(x, edge_index, batch, W, b):
    xp = jnp.zeros((NP, D), jnp.float32).at[:N].set(x)
    npad = EP - E
    row = jnp.concatenate(
        [edge_index[0], (N + (jnp.arange(npad, dtype=jnp.int32) % 16))]
    ).reshape(NCROW, CH)
    col = jnp.concatenate(
        [edge_index[1], jnp.zeros((npad,), jnp.int32)]).reshape(NCROW, CH)
    e2 = jnp.stack([col, row], axis=1)
    batch2 = jnp.full((1, NP), G, jnp.int32).at[0, :N].set(batch)

    ones_ch = jnp.ones((CH,), jnp.float32)
    zeros1 = jnp.zeros((RPT,), jnp.float32)
    zeros2 = jnp.zeros((ZR, DH), jnp.float32)
    b2 = b.reshape(1, D)

    deg = _sc_degree(e2, ones_ch, zeros1)
    dg0 = deg[0].reshape(NP, 1)
    dg1 = deg[1].reshape(NP, 1)
    dva, dvb, ya, yb = _tc_prep(dg0, dg1, xp)

    txs = [xp[:, :DH], xp[:, DH:]]
    txpa = txpb = None
    for _ in range(1, K):
        sa = _sc_spmm(ya, e2, zeros2)
        sb = _sc_spmm(yb, e2, zeros2)
        txa, ya = _tc_combine(sa, dva, txpa)
        txb, yb = _tc_combine(sb, dvb, txpb)
        txpa, txpb = txs[-2], txs[-1]
        txs.extend([txa, txb])

    return _tc_final(txs, W, b2, batch2)
